# Initial kernel scaffold; baseline (speedup 1.0000x reference)
import jax
import jax.numpy as jnp
from jax.experimental import pallas as pl


def kernel(xyz, points, t_embed, conv_w_0, time_w_0, time_b_0, bn_g_0, bn_b_0, conv_w_1, time_w_1, time_b_1, bn_g_1, bn_b_1, conv_w_2, time_w_2, time_b_2, bn_g_2, bn_b_2):
    raise NotImplementedError("write your pallas kernel here")


# speedup vs baseline: 4.1259x; 4.1259x over previous
import functools

import jax
import jax.numpy as jnp
from jax import lax
from jax.experimental import pallas as pl
from jax.experimental.pallas import tpu as pltpu
from jax.experimental.pallas import tpu_sc as plsc

_B = 16
_N = 2048
_S = 512
_K = 32
_RADIUS2 = 0.2 * 0.2
_EPS = 1e-5
_INV_SQRT2 = 0.7071067811865476


def _gelu(x):
    return 0.5 * x * (1.0 + lax.erf(x * _INV_SQRT2))


def _rows(v, r):
    ones = jnp.ones((r, 1), jnp.float32)
    return lax.dot_general(ones, v, (((1,), (0,)), ((), ())),
                           preferred_element_type=jnp.float32,
                           precision=lax.Precision.HIGHEST)


def _bdot(a, b, dims):
    return lax.dot_general(a.astype(jnp.bfloat16), b.astype(jnp.bfloat16),
                           (dims, ((), ())),
                           preferred_element_type=jnp.float32)


def _time_body(t_ref, w0_ref, b0_ref, w1_ref, b1_ref, w2_ref, b2_ref,
               o0_ref, o1_ref, o2_ref):
    t = _gelu(t_ref[...])
    for w_ref, b_ref, o_ref in ((w0_ref, b0_ref, o0_ref),
                                (w1_ref, b1_ref, o1_ref),
                                (w2_ref, b2_ref, o2_ref)):
        o_ref[...] = _bdot(t, w_ref[...], ((1,), (0,))) \
            + _rows(b_ref[...], t.shape[0])


def _time_mlp(t_embed, tw0, tb0, tw1, tb1, tw2, tb2):
    outs = [jax.ShapeDtypeStruct((_B, tw.shape[0]), jnp.float32)
            for tw in (tw0, tw1, tw2)]
    return pl.pallas_call(
        _time_body,
        out_shape=outs,
    )(t_embed, tw0.T, tb0.reshape(1, -1), tw1.T, tb1.reshape(1, -1),
      tw2.T, tb2.reshape(1, -1))


def _fps_body(xyz_ref, idx_ref, nx_ref):
    x = xyz_ref[0]
    iota_n = lax.broadcasted_iota(jnp.int32, (1, _N), 1)
    ones8 = jnp.ones((8, 1), dtype=jnp.float32)

    def body(i, carry):
        dist, far, cent, nx = carry
        far8 = lax.dot_general(ones8, far.astype(jnp.float32),
                               (((1,), (0,)), ((), ())),
                               preferred_element_type=jnp.float32,
                               precision=lax.Precision.HIGHEST)
        colmask = lax.broadcasted_iota(jnp.int32, (8, _S), 1) == i
        cent = jnp.where(colmask, jnp.broadcast_to(far8, (8, _S)), cent)
        onehot = (iota_n == far).astype(jnp.float32)
        c = jnp.sum(x * onehot, axis=1, keepdims=True)
        nx = jnp.where(colmask, jnp.broadcast_to(c, (8, _S)), nx)
        d = jnp.sum((x - jnp.broadcast_to(c, (8, _N))) ** 2, axis=0,
                    keepdims=True)
        dist = jnp.minimum(dist, d)
        far = jnp.argmax(dist, axis=1).astype(jnp.int32).reshape(1, 1)
        return dist, far, cent, nx

    dist0 = jnp.full((1, _N), 1e10, dtype=jnp.float32)
    far0 = jnp.zeros((1, 1), dtype=jnp.int32)
    cent0 = jnp.zeros((8, _S), dtype=jnp.float32)
    nx0 = jnp.zeros((8, _S), dtype=jnp.float32)
    _, _, cent, nx = lax.fori_loop(0, _S, body, (dist0, far0, cent0, nx0))
    idx_ref[0] = cent[0:1].astype(jnp.int32)
    nx_ref[0] = nx


def _fps(xyz_t8):
    return pl.pallas_call(
        _fps_body,
        grid=(_B,),
        in_specs=[pl.BlockSpec((1, 8, _N), lambda b: (b, 0, 0))],
        out_specs=[pl.BlockSpec((1, 1, _S), lambda b: (b, 0, 0)),
                   pl.BlockSpec((1, 8, _S), lambda b: (b, 0, 0))],
        out_shape=[jax.ShapeDtypeStruct((_B, 1, _S), jnp.int32),
                   jax.ShapeDtypeStruct((_B, 8, _S), jnp.float32)],
    )(xyz_t8)


def _ball_body(xyz_ref, nx_ref, idx_ref):
    x = xyz_ref[0]
    n = nx_ref[0]
    sq_x = jnp.sum(x * x, axis=0, keepdims=True)
    sq_n = jnp.sum(n * n, axis=0, keepdims=True)
    hp = dict(preferred_element_type=jnp.float32,
              precision=lax.Precision.HIGHEST)
    sqn_mat = lax.dot_general(sq_n, jnp.ones((1, _N), jnp.float32),
                              (((0,), (0,)), ((), ())), **hp)
    sqx_mat = lax.dot_general(jnp.ones((1, _S), jnp.float32), sq_x,
                              (((0,), (0,)), ((), ())), **hp)
    dn = _bdot(n, x, ((0,), (0,)))
    sqd = (sqn_mat + sqx_mat) - 2.0 * dn

    iota_n = jnp.broadcast_to(lax.broadcasted_iota(jnp.int32, (1, _N), 1),
                              (_S, _N))
    cur = jnp.where(sqd <= _RADIUS2, iota_n, _N)
    iota_k = lax.broadcasted_iota(jnp.int32, (_S, _K), 1)
    acc = jnp.zeros((_S, _K), dtype=jnp.int32)
    first = None
    for k in range(_K):
        m = jnp.min(cur, axis=1, keepdims=True)
        if first is None:
            first = m
            sel = m
        else:
            sel = jnp.where(m == _N, first, m)
        acc = jnp.where(iota_k == k, jnp.broadcast_to(sel, (_S, _K)), acc)
        cur = jnp.where(cur == m, _N, cur)
    idx_ref[0] = acc


def _ball(xyz_t8, nx8):
    return pl.pallas_call(
        _ball_body,
        grid=(_B,),
        in_specs=[pl.BlockSpec((1, 8, _N), lambda b: (b, 0, 0)),
                  pl.BlockSpec((1, 8, _S), lambda b: (b, 0, 0))],
        out_specs=pl.BlockSpec((1, _S, _K), lambda b: (b, 0, 0)),
        out_shape=jax.ShapeDtypeStruct((_B, _S, _K), jnp.int32),
    )(xyz_t8, nx8)


def _g_body(xyz_ref, pts_ref, wxyz_ref, wp_ref, g_ref):
    x = xyz_ref[0]
    p = pts_ref[0]
    gx = _bdot(x, wxyz_ref[...], ((0,), (0,)))
    gp = _bdot(p, wp_ref[...], ((0,), (0,)))
    g_ref[0] = gx + gp


def _g_project(xyz_t8, points, wxyz_pad, wp_t):
    c0 = wp_t.shape[1]
    return pl.pallas_call(
        _g_body,
        grid=(_B,),
        in_specs=[pl.BlockSpec((1, 8, _N), lambda b: (b, 0, 0)),
                  pl.BlockSpec((1, 128, _N), lambda b: (b, 0, 0)),
                  pl.BlockSpec((8, c0), lambda b: (0, 0)),
                  pl.BlockSpec((128, c0), lambda b: (0, 0))],
        out_specs=pl.BlockSpec((1, _N, c0), lambda b: (b, 0, 0)),
        out_shape=jax.ShapeDtypeStruct((_B, _N, c0), jnp.float32),
    )(xyz_t8, points, wxyz_pad, wp_t)


_SC_CHUNK = 128


def _sc_gather(table, idxg):
    rows, d = idxg.shape[0], table.shape[1]
    info = plsc.get_sparse_core_info()
    nw = info.num_cores * info.num_subcores
    rows_per_w = rows // nw
    n_chunks = rows_per_w // _SC_CHUNK
    mesh = plsc.VectorSubcoreMesh(core_axis_name="c", subcore_axis_name="s")

    @functools.partial(
        pl.kernel, mesh=mesh,
        out_type=jax.ShapeDtypeStruct((rows, d), jnp.float32),
        scratch_types=[
            pltpu.VMEM((_SC_CHUNK,), jnp.int32),
            pltpu.VMEM((_SC_CHUNK, d), jnp.float32),
            pltpu.SemaphoreType.DMA,
        ],
    )
    def gk(table_hbm, idx_hbm, out_hbm, idx_v, rows_v, sem):
        wid = lax.axis_index("s") * info.num_cores + lax.axis_index("c")
        base = wid * rows_per_w

        def body(i, carry):
            off = base + i * _SC_CHUNK
            pltpu.sync_copy(idx_hbm.at[pl.ds(off, _SC_CHUNK)], idx_v)
            pltpu.async_copy(table_hbm.at[idx_v], rows_v, sem).wait()
            pltpu.sync_copy(rows_v, out_hbm.at[pl.ds(off, _SC_CHUNK)])
            return carry

        lax.fori_loop(0, n_chunks, body, 0)

    return gk(table, idxg)


def _corr_body(nx_ref, wxyz_ref, t0_ref, corr_ref):
    n = nx_ref[0]
    proj = _bdot(n, wxyz_ref[...], ((0,), (0,)))
    corr_ref[0] = _rows(t0_ref[0], _S) - proj


def _corr(nx8, wxyz_pad, t0_3d):
    c0 = wxyz_pad.shape[1]
    return pl.pallas_call(
        _corr_body,
        grid=(_B,),
        in_specs=[pl.BlockSpec((1, 8, _S), lambda b: (b, 0, 0)),
                  pl.BlockSpec((8, c0), lambda b: (0, 0)),
                  pl.BlockSpec((1, 1, c0), lambda b: (b, 0, 0))],
        out_specs=pl.BlockSpec((1, _S, c0), lambda b: (b, 0, 0)),
        out_shape=jax.ShapeDtypeStruct((_B, _S, c0), jnp.float32),
    )(nx8, wxyz_pad, t0_3d)


_RB = 2048
_GB = _RB // _K


def _expand_mat():
    r = lax.broadcasted_iota(jnp.int32, (_RB, _GB), 0) // _K
    q = lax.broadcasted_iota(jnp.int32, (_RB, _GB), 1)
    return (r == q).astype(jnp.float32)


def _y0_block(g_ref, c_ref):
    e = _expand_mat()
    cexp = lax.dot_general(e, c_ref[0, 0], (((1,), (0,)), ((), ())),
                           preferred_element_type=jnp.float32,
                           precision=lax.Precision.HIGHEST)
    return g_ref[0, 0] + cexp


def _accum_stats(y, s1_ref, s2_ref):
    step = pl.program_id(0) * pl.num_programs(1) + pl.program_id(1)

    @pl.when(step == 0)
    def _():
        s1_ref[...] = jnp.zeros_like(s1_ref)
        s2_ref[...] = jnp.zeros_like(s2_ref)

    s1_ref[...] += jnp.sum(y, axis=0, keepdims=True)
    s2_ref[...] += jnp.sum(y * y, axis=0, keepdims=True)


def _stats0_body(g_ref, c_ref, s1_ref, s2_ref):
    _accum_stats(_y0_block(g_ref, c_ref), s1_ref, s2_ref)


def _stats0(gg4, corr4):
    c0 = gg4.shape[3]
    return pl.pallas_call(
        _stats0_body,
        grid=(_B, 8),
        in_specs=[pl.BlockSpec((1, 1, _RB, c0), lambda b, j: (b, j, 0, 0)),
                  pl.BlockSpec((1, 1, _GB, c0), lambda b, j: (b, j, 0, 0))],
        out_specs=[pl.BlockSpec((1, c0), lambda b, j: (0, 0)),
                   pl.BlockSpec((1, c0), lambda b, j: (0, 0))],
        out_shape=[jax.ShapeDtypeStruct((1, c0), jnp.float32),
                   jax.ShapeDtypeStruct((1, c0), jnp.float32)],
    )(gg4, corr4)


def _layer1_body(g_ref, c_ref, sc_ref, sh_ref, w_ref, t_ref,
                 y_ref, s1_ref, s2_ref):
    y0 = _y0_block(g_ref, c_ref)
    z0 = _gelu(y0 * _rows(sc_ref[...], _RB) + _rows(sh_ref[...], _RB))
    y1 = _bdot(z0, w_ref[...], ((1,), (0,))) + _rows(t_ref[0], _RB)
    y_ref[0, 0] = y1
    _accum_stats(y1, s1_ref, s2_ref)


def _layer1(gg4, corr4, scale0, shift0, w1_t, t1_3d):
    c0, c1 = gg4.shape[3], w1_t.shape[1]
    return pl.pallas_call(
        _layer1_body,
        grid=(_B, 8),
        in_specs=[pl.BlockSpec((1, 1, _RB, c0), lambda b, j: (b, j, 0, 0)),
                  pl.BlockSpec((1, 1, _GB, c0), lambda b, j: (b, j, 0, 0)),
                  pl.BlockSpec((1, c0), lambda b, j: (0, 0)),
                  pl.BlockSpec((1, c0), lambda b, j: (0, 0)),
                  pl.BlockSpec((c0, c1), lambda b, j: (0, 0)),
                  pl.BlockSpec((1, 1, c1), lambda b, j: (b, 0, 0))],
        out_specs=[pl.BlockSpec((1, 1, _RB, c1), lambda b, j: (b, j, 0, 0)),
                   pl.BlockSpec((1, c1), lambda b, j: (0, 0)),
                   pl.BlockSpec((1, c1), lambda b, j: (0, 0))],
        out_shape=[jax.ShapeDtypeStruct((_B, 8, _RB, c1), jnp.float32),
                   jax.ShapeDtypeStruct((1, c1), jnp.float32),
                   jax.ShapeDtypeStruct((1, c1), jnp.float32)],
    )(gg4, corr4, scale0, shift0, w1_t, t1_3d)


def _layer2_body(y1_ref, sc_ref, sh_ref, w_ref, t_ref, y_ref, s1_ref, s2_ref):
    z1 = _gelu(y1_ref[0, 0] * _rows(sc_ref[...], _RB) + _rows(sh_ref[...], _RB))
    y2 = _bdot(z1, w_ref[...], ((1,), (0,))) + _rows(t_ref[0], _RB)
    y_ref[0, 0] = y2
    _accum_stats(y2, s1_ref, s2_ref)


def _layer2(y1, scale1, shift1, w2_t, t2_3d):
    c1, c2 = y1.shape[3], w2_t.shape[1]
    return pl.pallas_call(
        _layer2_body,
        grid=(_B, 8),
        in_specs=[pl.BlockSpec((1, 1, _RB, c1), lambda b, j: (b, j, 0, 0)),
                  pl.BlockSpec((1, c1), lambda b, j: (0, 0)),
                  pl.BlockSpec((1, c1), lambda b, j: (0, 0)),
                  pl.BlockSpec((c1, c2), lambda b, j: (0, 0)),
                  pl.BlockSpec((1, 1, c2), lambda b, j: (b, 0, 0))],
        out_specs=[pl.BlockSpec((1, 1, _RB, c2), lambda b, j: (b, j, 0, 0)),
                   pl.BlockSpec((1, c2), lambda b, j: (0, 0)),
                   pl.BlockSpec((1, c2), lambda b, j: (0, 0))],
        out_shape=[jax.ShapeDtypeStruct((_B, 8, _RB, c2), jnp.float32),
                   jax.ShapeDtypeStruct((1, c2), jnp.float32),
                   jax.ShapeDtypeStruct((1, c2), jnp.float32)],
    )(y1, scale1, shift1, w2_t, t2_3d)


def _final_body(y2_ref, sc_ref, sh_ref, o_ref):
    z = _gelu(y2_ref[0, 0] * _rows(sc_ref[...], _RB)
              + _rows(sh_ref[...], _RB))
    z3 = z.reshape(_GB, _K, z.shape[1])
    acc = z3[:, 0, :]
    for k in range(1, _K):
        acc = jnp.maximum(acc, z3[:, k, :])
    o_ref[0, 0] = acc


def _final(y2, scale2, shift2):
    c2 = y2.shape[3]
    return pl.pallas_call(
        _final_body,
        grid=(_B, 8),
        in_specs=[pl.BlockSpec((1, 1, _RB, c2), lambda b, j: (b, j, 0, 0)),
                  pl.BlockSpec((1, c2), lambda b, j: (0, 0)),
                  pl.BlockSpec((1, c2), lambda b, j: (0, 0))],
        out_specs=pl.BlockSpec((1, 1, _GB, c2), lambda b, j: (b, j, 0, 0)),
        out_shape=jax.ShapeDtypeStruct((_B, 8, _GB, c2), jnp.float32),
    )(y2, scale2, shift2)


def _bn_params(s1, s2, gamma, beta):
    n = float(_B * _S * _K)
    mean = s1 / n
    var = s2 / n - mean * mean
    rstd = 1.0 / jnp.sqrt(var + _EPS)
    scale = gamma.reshape(1, -1) * rstd
    shift = beta.reshape(1, -1) - mean * scale
    return scale, shift


def kernel(xyz, points, t_embed,
           conv_w_0, time_w_0, time_b_0, bn_g_0, bn_b_0,
           conv_w_1, time_w_1, time_b_1, bn_g_1, bn_b_1,
           conv_w_2, time_w_2, time_b_2, bn_g_2, bn_b_2):
    xyz_t8 = jnp.pad(jnp.transpose(xyz, (0, 2, 1)), ((0, 0), (0, 5), (0, 0)))

    t0, t1, t2 = _time_mlp(t_embed, time_w_0, time_b_0, time_w_1, time_b_1,
                           time_w_2, time_b_2)

    fps3, nx8 = _fps(xyz_t8)
    idx = _ball(xyz_t8, nx8)

    wxyz_pad = jnp.pad(conv_w_0[:, :3].T, ((0, 5), (0, 0)))
    wp_t = conv_w_0[:, 3:].T
    g = _g_project(xyz_t8, points, wxyz_pad, wp_t)

    idxg = (idx + (jnp.arange(_B, dtype=jnp.int32) * _N)[:, None, None])
    gg = _sc_gather(g.reshape(_B * _N, -1), idxg.reshape(-1))

    corr = _corr(nx8, wxyz_pad, t0.reshape(_B, 1, -1))

    gg4 = gg.reshape(_B, 8, _RB, -1)
    corr4 = corr.reshape(_B, 8, _GB, -1)

    s1, s2 = _stats0(gg4, corr4)
    scale0, shift0 = _bn_params(s1, s2, bn_g_0, bn_b_0)

    y1, s1_1, s2_1 = _layer1(gg4, corr4, scale0, shift0, conv_w_1.T,
                             t1.reshape(_B, 1, -1))
    scale1, shift1 = _bn_params(s1_1, s2_1, bn_g_1, bn_b_1)

    y2, s1_2, s2_2 = _layer2(y1, scale1, shift1, conv_w_2.T,
                             t2.reshape(_B, 1, -1))
    scale2, shift2 = _bn_params(s1_2, s2_2, bn_g_2, bn_b_2)

    npool = _final(y2, scale2, shift2)

    new_xyz = jnp.transpose(nx8[:, :3, :], (0, 2, 1))
    new_points = jnp.transpose(npool.reshape(_B, _S, -1), (0, 2, 1))
    return (new_xyz, new_points)
